# baseline (speedup 1.0000x reference)
import jax
import jax.numpy as jnp
from jax import lax
from jax.experimental import pallas as pl
from jax.experimental.pallas import tpu as pltpu
from jax.experimental.pallas import tpu_sc as plsc

N = 10000
E = 320000
D = 128
NS = 10240
EP = 327680
HC = EP // 128 // 16
EM = E // 32

_MESH = plsc.VectorSubcoreMesh(core_axis_name="c", subcore_axis_name="s")
_SC_PARAMS = pltpu.CompilerParams(needs_layout_passes=False)


def _sc_body(col2_hbm, zero_hbm, ones_hbm, ei_hbm, ea_hbm, pq_hbm, out_hbm,
             colv, onesv, degv, pvv, qvv, uv, pcv, rv, cv, eav, ov, deg_sh,
             semH, semA, semB):
    c = lax.axis_index("c")
    s = lax.axis_index("s")
    wid = c * 16 + s
    base = wid * EM

    h1 = pltpu.async_copy(col2_hbm.at[pl.ds(s * HC, HC)], colv, semH)
    h2 = pltpu.async_copy(ones_hbm, onesv, semH)
    a1 = pltpu.async_copy(pq_hbm.at[pl.ds(0, N)], pvv, semA)
    b1 = pltpu.async_copy(pq_hbm.at[pl.ds(N, N)], qvv, semB)
    b2 = pltpu.async_copy(ei_hbm.at[pl.ds(base, EM)], rv, semB)
    b3 = pltpu.async_copy(ei_hbm.at[pl.ds(E + base, EM)], cv, semB)
    b4 = pltpu.async_copy(ea_hbm.at[pl.ds(base, EM)], eav, semB)

    @pl.when(s == 0)
    def _():
        pltpu.sync_copy(zero_hbm, deg_sh)

    h1.wait()
    h2.wait()
    plsc.subcore_barrier()
    descs = [pltpu.async_copy(onesv, deg_sh.at[colv.at[j]], add=True, sem=semH)
             for j in range(HC)]
    for d in descs:
        d.wait()
    plsc.subcore_barrier()

    pltpu.sync_copy(deg_sh, degv)
    a1.wait()

    @plsc.parallel_loop(0, N, step=16, unroll=4)
    def _table(i):
        sl = pl.ds(i, 16)
        d = degv[sl]
        half = d * 0.5
        y = lax.bitcast_convert_type(
            jnp.int32(0x5F3759DF) - (lax.bitcast_convert_type(d, jnp.int32) >> 1),
            jnp.float32)
        y = y * (1.5 - half * y * y)
        y = y * (1.5 - half * y * y)
        y = y * (1.5 - half * y * y)
        u = jnp.where(d == 0.0, jnp.full((16,), jnp.inf, jnp.float32), y)
        uv[sl] = u
        pcv[sl] = u * pvv[sl]

    b1.wait()
    b2.wait()
    b3.wait()
    b4.wait()

    @plsc.parallel_loop(0, EM, step=16, unroll=4)
    def _edge(i):
        sl = pl.ds(i, 16)
        ir = rv[sl]
        ic = cv[sl]
        ur = plsc.load_gather(uv, [ir])
        qr = plsc.load_gather(qvv, [ir])
        pcc = plsc.load_gather(pcv, [ic])
        ov[sl] = eav[sl] * (ur * pcc + qr)

    pltpu.sync_copy(ov, out_hbm.at[pl.ds(base, EM)])


_sc_fused = pl.kernel(
    _sc_body,
    out_type=jax.ShapeDtypeStruct((E,), jnp.float32),
    mesh=_MESH,
    compiler_params=_SC_PARAMS,
    scratch_types=[
        pltpu.VMEM((HC, 128), jnp.int32),
        pltpu.VMEM((128,), jnp.float32),
        pltpu.VMEM((NS,), jnp.float32),
        pltpu.VMEM((N,), jnp.float32),
        pltpu.VMEM((N,), jnp.float32),
        pltpu.VMEM((NS,), jnp.float32),
        pltpu.VMEM((N,), jnp.float32),
        pltpu.VMEM((EM,), jnp.int32),
        pltpu.VMEM((EM,), jnp.int32),
        pltpu.VMEM((EM,), jnp.float32),
        pltpu.VMEM((EM,), jnp.float32),
        pltpu.VMEM_SHARED((NS,), jnp.float32),
        pltpu.SemaphoreType.DMA,
        pltpu.SemaphoreType.DMA,
        pltpu.SemaphoreType.DMA,
    ],
)


def _mv_body(x_ref, w_ref, b_ref, o_ref):
    o_ref[...] = jnp.maximum(
        lax.dot_general(w_ref[...], x_ref[...],
                        (((1,), (1,)), ((), ())),
                        preferred_element_type=jnp.float32)
        + b_ref[...], 0.0)


def _matvec(x, w, b):
    return pl.pallas_call(
        _mv_body,
        out_shape=jax.ShapeDtypeStruct((2, N), jnp.float32),
    )(x, w, b)


def kernel(x, edge_index, edge_attr, p_w, p_b, q_w, q_b):
    ei = edge_index.astype(jnp.int32)
    col2 = jnp.concatenate(
        [ei[1], jnp.full((EP - E,), NS - 1, jnp.int32)]).reshape(EP // 128, 128)
    zero = jnp.zeros((NS,), jnp.float32)
    ones = jnp.ones((128,), jnp.float32)

    w = jnp.concatenate([p_w, q_w], axis=0)
    b = jnp.concatenate([p_b, q_b]).reshape(2, 1)
    pq = _matvec(x, w, b).reshape(-1)

    out = _sc_fused(col2, zero, ones, ei.reshape(-1), edge_attr, pq)
    return (edge_index, out)


# speedup vs baseline: 1.1642x; 1.1642x over previous
import jax
import jax.numpy as jnp
from jax import lax
from jax.experimental import pallas as pl
from jax.experimental.pallas import tpu as pltpu
from jax.experimental.pallas import tpu_sc as plsc

N = 10000
E = 320000
D = 128
NS = 10112
HC = 80
EP = 32 * HC * 128
EM = E // 32

_MESH = plsc.VectorSubcoreMesh(core_axis_name="c", subcore_axis_name="s")
_SC_PARAMS = pltpu.CompilerParams(needs_layout_passes=False)


def _hist_body(col2_hbm, zero_hbm, ones_hbm, deg_hbm, colv, onesv, deg_sh, sem):
    c = lax.axis_index("c")
    s = lax.axis_index("s")
    wid = c * 16 + s

    with jax.named_scope("hist_stage"):
        d1 = pltpu.async_copy(col2_hbm.at[pl.ds(wid * HC, HC)], colv, sem)
        d2 = pltpu.async_copy(ones_hbm, onesv, sem)

        @pl.when(s == 0)
        def _():
            pltpu.sync_copy(zero_hbm, deg_sh)

        d1.wait()
        d2.wait()
        plsc.subcore_barrier()
    with jax.named_scope("hist_scatter"):
        descs = [pltpu.async_copy(onesv, deg_sh.at[colv.at[j]], add=True,
                                  sem=sem)
                 for j in range(HC)]
        for d in descs:
            d.wait()
        plsc.subcore_barrier()

    with jax.named_scope("hist_wb"):
        @pl.when(s == 0)
        def _():
            pltpu.sync_copy(deg_sh, deg_hbm.at[pl.ds(c * NS, NS)])


_hist = pl.kernel(
    _hist_body,
    out_type=jax.ShapeDtypeStruct((2 * NS,), jnp.float32),
    mesh=_MESH,
    compiler_params=_SC_PARAMS,
    scratch_types=[
        pltpu.VMEM((HC, 128), jnp.int32),
        pltpu.VMEM((128,), jnp.float32),
        pltpu.VMEM_SHARED((NS,), jnp.float32),
        pltpu.SemaphoreType.DMA,
    ],
)


def _map_body(ei_hbm, ea_hbm, deg_hbm, pq_hbm, out_hbm,
              d0v, d1v, pvv, qvv, uv, pcv, rv, cv, eav, ov, semA, semB):
    c = lax.axis_index("c")
    s = lax.axis_index("s")
    wid = c * 16 + s
    base = wid * EM

    with jax.named_scope("map_stage"):
        a1 = pltpu.async_copy(deg_hbm.at[pl.ds(0, N)], d0v, semA)
        a2 = pltpu.async_copy(deg_hbm.at[pl.ds(NS, N)], d1v, semA)
        a3 = pltpu.async_copy(pq_hbm.at[pl.ds(0, N)], pvv, semA)
        b1 = pltpu.async_copy(ei_hbm.at[pl.ds(base, EM)], rv, semB)
        b2 = pltpu.async_copy(ei_hbm.at[pl.ds(E + base, EM)], cv, semB)
        b3 = pltpu.async_copy(ea_hbm.at[pl.ds(base, EM)], eav, semB)
        b4 = pltpu.async_copy(pq_hbm.at[pl.ds(N, N)], qvv, semB)
        a1.wait()
        a2.wait()
        a3.wait()

    with jax.named_scope("map_table"):
        @plsc.parallel_loop(0, N, step=16, unroll=4)
        def _table(i):
            sl = pl.ds(i, 16)
            d = d0v[sl] + d1v[sl]
            half = d * 0.5
            y = lax.bitcast_convert_type(
                jnp.int32(0x5F3759DF)
                - (lax.bitcast_convert_type(d, jnp.int32) >> 1),
                jnp.float32)
            y = y * (1.5 - half * y * y)
            y = y * (1.5 - half * y * y)
            y = y * (1.5 - half * y * y)
            u = jnp.where(d == 0.0, jnp.full((16,), jnp.inf, jnp.float32), y)
            uv[sl] = u
            pcv[sl] = u * pvv[sl]

    with jax.named_scope("map_wait_edges"):
        b1.wait()
        b2.wait()
        b3.wait()
        b4.wait()

    with jax.named_scope("map_edges"):
        @plsc.parallel_loop(0, EM, step=16, unroll=4)
        def _edge(i):
            sl = pl.ds(i, 16)
            ir = rv[sl]
            ic = cv[sl]
            ur = plsc.load_gather(uv, [ir])
            qr = plsc.load_gather(qvv, [ir])
            pcc = plsc.load_gather(pcv, [ic])
            ov[sl] = eav[sl] * (ur * pcc + qr)

    with jax.named_scope("map_wb"):
        pltpu.sync_copy(ov, out_hbm.at[pl.ds(base, EM)])


_map = pl.kernel(
    _map_body,
    out_type=jax.ShapeDtypeStruct((E,), jnp.float32),
    mesh=_MESH,
    compiler_params=_SC_PARAMS,
    scratch_types=[
        pltpu.VMEM((N,), jnp.float32),
        pltpu.VMEM((N,), jnp.float32),
        pltpu.VMEM((N,), jnp.float32),
        pltpu.VMEM((N,), jnp.float32),
        pltpu.VMEM((N,), jnp.float32),
        pltpu.VMEM((N,), jnp.float32),
        pltpu.VMEM((EM,), jnp.int32),
        pltpu.VMEM((EM,), jnp.int32),
        pltpu.VMEM((EM,), jnp.float32),
        pltpu.VMEM((EM,), jnp.float32),
        pltpu.SemaphoreType.DMA,
        pltpu.SemaphoreType.DMA,
    ],
)


def _mv_body(x_ref, w_ref, b_ref, o_ref):
    o_ref[...] = jnp.maximum(
        lax.dot_general(w_ref[...], x_ref[...],
                        (((1,), (1,)), ((), ())),
                        preferred_element_type=jnp.float32)
        + b_ref[...], 0.0)


def _matvec(x, w, b):
    return pl.pallas_call(
        _mv_body,
        out_shape=jax.ShapeDtypeStruct((2, N), jnp.float32),
    )(x, w, b)


def kernel(x, edge_index, edge_attr, p_w, p_b, q_w, q_b):
    ei = edge_index.astype(jnp.int32)
    col2 = jnp.concatenate(
        [ei[1], jnp.full((EP - E,), NS - 1, jnp.int32)]).reshape(EP // 128, 128)
    zero = jnp.zeros((NS,), jnp.float32)
    ones = jnp.ones((128,), jnp.float32)

    w = jnp.concatenate([p_w, q_w], axis=0)
    b = jnp.concatenate([p_b, q_b]).reshape(2, 1)
    pq = _matvec(x, w, b).reshape(-1)

    deg = _hist(col2, zero, ones)
    out = _map(ei.reshape(-1), edge_attr, deg, pq)
    return (edge_index, out)


# speedup vs baseline: 1.5432x; 1.3255x over previous
import jax
import jax.numpy as jnp
from jax import lax
from jax.experimental import pallas as pl
from jax.experimental.pallas import tpu as pltpu
from jax.experimental.pallas import tpu_sc as plsc

N = 10000
E = 320000
D = 128
NS = 10240
EH = E // 16
HC = EH // 128
HT = EH - HC * 128
EM = E // 32
NZ = NS // 16

_MESH = plsc.VectorSubcoreMesh(core_axis_name="c", subcore_axis_name="s")
_SC_PARAMS = pltpu.CompilerParams(needs_layout_passes=False)


def _sc_body(ei_hbm, ea_hbm, pq_hbm, out_hbm,
             colh, zv, onesv, degv, pvv, qvv, uv, pcv, rv, cv, eav, ov,
             deg_sh, semH, semA, semB):
    c = lax.axis_index("c")
    s = lax.axis_index("s")
    wid = c * 16 + s
    base = wid * EM

    with jax.named_scope("stage"):
        h1 = pltpu.async_copy(ei_hbm.at[pl.ds(E + s * EH, EH)], colh, semH)
        a1 = pltpu.async_copy(pq_hbm.at[pl.ds(0, N)], pvv, semA)
        b1 = pltpu.async_copy(pq_hbm.at[pl.ds(N, N)], qvv, semB)
        b2 = pltpu.async_copy(ei_hbm.at[pl.ds(base, EM)], rv, semB)
        b3 = pltpu.async_copy(ei_hbm.at[pl.ds(E + base, EM)], cv, semB)
        b4 = pltpu.async_copy(ea_hbm.at[pl.ds(base, EM)], eav, semB)

        for k in range(NZ // 16):
            zv[pl.ds(k * 16, 16)] = jnp.zeros((16,), jnp.float32)
        for k in range(8):
            onesv[pl.ds(k * 16, 16)] = jnp.full((16,), 1.0, jnp.float32)
        pltpu.sync_copy(zv, deg_sh.at[pl.ds(s * NZ, NZ)])

        h1.wait()
        plsc.subcore_barrier()

    with jax.named_scope("hist_scatter"):
        descs = [pltpu.async_copy(onesv.at[pl.ds(0, 128)],
                                  deg_sh.at[colh.at[pl.ds(j * 128, 128)]],
                                  add=True, sem=semH)
                 for j in range(HC)]
        descs.append(pltpu.async_copy(onesv.at[pl.ds(0, HT)],
                                      deg_sh.at[colh.at[pl.ds(HC * 128, HT)]],
                                      add=True, sem=semH))
        for d in descs:
            d.wait()
        plsc.subcore_barrier()

    with jax.named_scope("deg_fetch"):
        pltpu.sync_copy(deg_sh, degv)
        a1.wait()

    with jax.named_scope("table"):
        @plsc.parallel_loop(0, N, step=16, unroll=4)
        def _table(i):
            sl = pl.ds(i, 16)
            d = degv[sl]
            half = d * 0.5
            y = lax.bitcast_convert_type(
                jnp.int32(0x5F3759DF)
                - (lax.bitcast_convert_type(d, jnp.int32) >> 1),
                jnp.float32)
            y = y * (1.5 - half * y * y)
            y = y * (1.5 - half * y * y)
            y = y * (1.5 - half * y * y)
            u = jnp.where(d == 0.0, jnp.full((16,), jnp.inf, jnp.float32), y)
            uv[sl] = u
            pcv[sl] = u * pvv[sl]

    with jax.named_scope("wait_edges"):
        b1.wait()
        b2.wait()
        b3.wait()
        b4.wait()

    with jax.named_scope("edges"):
        @plsc.parallel_loop(0, EM, step=16, unroll=4)
        def _edge(i):
            sl = pl.ds(i, 16)
            ir = rv[sl]
            ic = cv[sl]
            ur = plsc.load_gather(uv, [ir])
            qr = plsc.load_gather(qvv, [ir])
            pcc = plsc.load_gather(pcv, [ic])
            ov[sl] = eav[sl] * (ur * pcc + qr)

    with jax.named_scope("wb"):
        pltpu.sync_copy(ov, out_hbm.at[pl.ds(base, EM)])


_sc_fused = pl.kernel(
    _sc_body,
    out_type=jax.ShapeDtypeStruct((E,), jnp.float32),
    mesh=_MESH,
    compiler_params=_SC_PARAMS,
    scratch_types=[
        pltpu.VMEM((EH,), jnp.int32),
        pltpu.VMEM((NZ,), jnp.float32),
        pltpu.VMEM((128,), jnp.float32),
        pltpu.VMEM((NS,), jnp.float32),
        pltpu.VMEM((N,), jnp.float32),
        pltpu.VMEM((N,), jnp.float32),
        pltpu.VMEM((N,), jnp.float32),
        pltpu.VMEM((N,), jnp.float32),
        pltpu.VMEM((EM,), jnp.int32),
        pltpu.VMEM((EM,), jnp.int32),
        pltpu.VMEM((EM,), jnp.float32),
        pltpu.VMEM((EM,), jnp.float32),
        pltpu.VMEM_SHARED((NS,), jnp.float32),
        pltpu.SemaphoreType.DMA,
        pltpu.SemaphoreType.DMA,
        pltpu.SemaphoreType.DMA,
    ],
)


def _mv_body(x_ref, pw_ref, qw_ref, pb_ref, qb_ref, o_ref):
    w = jnp.concatenate([pw_ref[...], qw_ref[...]], axis=0)
    b = jnp.concatenate([pb_ref[...], qb_ref[...]], axis=0)
    res = jnp.maximum(
        lax.dot_general(w, x_ref[...], (((1,), (1,)), ((), ())),
                        preferred_element_type=jnp.float32) + b, 0.0)
    o_ref[pl.ds(0, N)] = res[0, :]
    o_ref[pl.ds(N, N)] = res[1, :]


def _matvec(x, p_w, q_w, p_b, q_b):
    return pl.pallas_call(
        _mv_body,
        out_shape=jax.ShapeDtypeStruct((2 * N,), jnp.float32),
    )(x, p_w, q_w, p_b, q_b)


def kernel(x, edge_index, edge_attr, p_w, p_b, q_w, q_b):
    eif = edge_index.astype(jnp.int32).reshape(-1)
    pq = _matvec(x, p_w, q_w, p_b.reshape(1, 1), q_b.reshape(1, 1))
    out = _sc_fused(eif, edge_attr, pq)
    return (edge_index, out)


# speedup vs baseline: 1.7551x; 1.1373x over previous
import jax
import jax.numpy as jnp
from jax import lax
from jax.experimental import pallas as pl
from jax.experimental.pallas import tpu as pltpu
from jax.experimental.pallas import tpu_sc as plsc

N = 10000
E = 320000
D = 128
NS = 10240
EH = E // 16
HC = EH // 128
HT = EH - HC * 128
EM = E // 32
NZ = NS // 16

_MESH = plsc.VectorSubcoreMesh(core_axis_name="c", subcore_axis_name="s")
_SC_PARAMS = pltpu.CompilerParams(needs_layout_passes=False)


def _sc_body(ei_hbm, ea_hbm, pq_hbm, out_hbm,
             colh, zv, onesv, degv, pvv, qvv, uv, pcv, rv, cv, eav, ov,
             deg_sh, semH, semA, semB):
    c = lax.axis_index("c")
    s = lax.axis_index("s")
    wid = c * 16 + s
    base = wid * EM

    with jax.named_scope("stage"):
        h1 = pltpu.async_copy(ei_hbm.at[pl.ds(E + s * EH, EH)], colh, semH)

        for k in range(NZ // 16):
            zv[pl.ds(k * 16, 16)] = jnp.zeros((16,), jnp.float32)
        for k in range(8):
            onesv[pl.ds(k * 16, 16)] = jnp.full((16,), 1.0, jnp.float32)
        pltpu.sync_copy(zv, deg_sh.at[pl.ds(s * NZ, NZ)])

        h1.wait()
        plsc.subcore_barrier()

    with jax.named_scope("hist_scatter"):
        def _fire(j, carry):
            pltpu.async_copy(onesv.at[pl.ds(0, 128)],
                             deg_sh.at[colh.at[pl.ds(j * 128, 128)]],
                             add=True, sem=semH)
            return carry

        lax.fori_loop(0, HC, _fire, 0)
        pltpu.async_copy(onesv.at[pl.ds(0, HT)],
                         deg_sh.at[colh.at[pl.ds(HC * 128, HT)]],
                         add=True, sem=semH)

        a1 = pltpu.async_copy(pq_hbm.at[pl.ds(0, N)], pvv, semA)
        b1 = pltpu.async_copy(pq_hbm.at[pl.ds(N, N)], qvv, semB)
        b2 = pltpu.async_copy(ei_hbm.at[pl.ds(base, EM)], rv, semB)
        b3 = pltpu.async_copy(ei_hbm.at[pl.ds(E + base, EM)], cv, semB)
        b4 = pltpu.async_copy(ea_hbm.at[pl.ds(base, EM)], eav, semB)

        def _drain(j, carry):
            pltpu.make_async_copy(onesv.at[pl.ds(0, 128)],
                                  deg_sh.at[colh.at[pl.ds(j * 128, 128)]],
                                  semH).wait()
            return carry

        lax.fori_loop(0, HC, _drain, 0)
        pltpu.make_async_copy(onesv.at[pl.ds(0, HT)],
                              deg_sh.at[colh.at[pl.ds(HC * 128, HT)]],
                              semH).wait()
        plsc.subcore_barrier()

    with jax.named_scope("deg_fetch"):
        pltpu.sync_copy(deg_sh, degv)
        a1.wait()

    with jax.named_scope("table"):
        @plsc.parallel_loop(0, N, step=16, unroll=4)
        def _table(i):
            sl = pl.ds(i, 16)
            d = degv[sl]
            half = d * 0.5
            y = lax.bitcast_convert_type(
                jnp.int32(0x5F3759DF)
                - (lax.bitcast_convert_type(d, jnp.int32) >> 1),
                jnp.float32)
            y = y * (1.5 - half * y * y)
            y = y * (1.5 - half * y * y)
            y = y * (1.5 - half * y * y)
            u = jnp.where(d == 0.0, jnp.full((16,), jnp.inf, jnp.float32), y)
            uv[sl] = u
            pcv[sl] = u * pvv[sl]

    with jax.named_scope("wait_edges"):
        b1.wait()
        b2.wait()
        b3.wait()
        b4.wait()

    with jax.named_scope("edges"):
        @plsc.parallel_loop(0, EM, step=16, unroll=4)
        def _edge(i):
            sl = pl.ds(i, 16)
            ir = rv[sl]
            ic = cv[sl]
            ur = plsc.load_gather(uv, [ir])
            qr = plsc.load_gather(qvv, [ir])
            pcc = plsc.load_gather(pcv, [ic])
            ov[sl] = eav[sl] * (ur * pcc + qr)

    with jax.named_scope("wb"):
        pltpu.sync_copy(ov, out_hbm.at[pl.ds(base, EM)])


_sc_fused = pl.kernel(
    _sc_body,
    out_type=jax.ShapeDtypeStruct((E,), jnp.float32),
    mesh=_MESH,
    compiler_params=_SC_PARAMS,
    scratch_types=[
        pltpu.VMEM((EH,), jnp.int32),
        pltpu.VMEM((NZ,), jnp.float32),
        pltpu.VMEM((128,), jnp.float32),
        pltpu.VMEM((NS,), jnp.float32),
        pltpu.VMEM((N,), jnp.float32),
        pltpu.VMEM((N,), jnp.float32),
        pltpu.VMEM((N,), jnp.float32),
        pltpu.VMEM((N,), jnp.float32),
        pltpu.VMEM((EM,), jnp.int32),
        pltpu.VMEM((EM,), jnp.int32),
        pltpu.VMEM((EM,), jnp.float32),
        pltpu.VMEM((EM,), jnp.float32),
        pltpu.VMEM_SHARED((NS,), jnp.float32),
        pltpu.SemaphoreType.DMA,
        pltpu.SemaphoreType.DMA,
        pltpu.SemaphoreType.DMA,
    ],
)


def _mv_body(x_ref, pw_ref, qw_ref, pb_ref, qb_ref, ei_ref, o_ref, oe_ref):
    w = jnp.concatenate([pw_ref[...], qw_ref[...]], axis=0)
    b = jnp.concatenate([pb_ref[...], qb_ref[...]], axis=0)
    res = jnp.maximum(
        lax.dot_general(w, x_ref[...], (((1,), (1,)), ((), ())),
                        preferred_element_type=jnp.float32) + b, 0.0)
    o_ref[pl.ds(0, N)] = res[0, :]
    o_ref[pl.ds(N, N)] = res[1, :]
    oe_ref[pl.ds(0, E)] = ei_ref[0, :]
    oe_ref[pl.ds(E, E)] = ei_ref[1, :]


def _prep(x, p_w, q_w, p_b, q_b, ei):
    return pl.pallas_call(
        _mv_body,
        out_shape=[jax.ShapeDtypeStruct((2 * N,), jnp.float32),
                   jax.ShapeDtypeStruct((2 * E,), jnp.int32)],
    )(x, p_w, q_w, p_b, q_b, ei)


def kernel(x, edge_index, edge_attr, p_w, p_b, q_w, q_b):
    ei = edge_index.astype(jnp.int32)
    pq, eif = _prep(x, p_w, q_w, p_b.reshape(1, 1), q_b.reshape(1, 1), ei)
    out = _sc_fused(eif, edge_attr, pq)
    return (edge_index, out)


# speedup vs baseline: 1.7839x; 1.0164x over previous
import jax
import jax.numpy as jnp
from jax import lax
from jax.experimental import pallas as pl
from jax.experimental.pallas import tpu as pltpu
from jax.experimental.pallas import tpu_sc as plsc

N = 10000
E = 320000
D = 128
NS = 10240
EM = E // 32
HC = EM // 128
HT = EM - HC * 128
NZ = NS // 16

_MESH = plsc.VectorSubcoreMesh(core_axis_name="c", subcore_axis_name="s")
_SC_PARAMS = pltpu.CompilerParams(needs_layout_passes=False)


def _sc_body(ei_hbm, ea_hbm, pq_hbm, out_hbm, degp_hbm,
             zv, onesv, degv, d2v, pvv, qvv, uv, pcv, rv, cv, eav, ov,
             deg_sh, semH, semA, semB, bsem):
    c = lax.axis_index("c")
    s = lax.axis_index("s")
    wid = c * 16 + s
    base = wid * EM

    with jax.named_scope("stage"):
        h1 = pltpu.async_copy(ei_hbm.at[pl.ds(E + base, EM)], cv, semH)
        a1 = pltpu.async_copy(pq_hbm.at[pl.ds(0, N)], pvv, semA)
        b1 = pltpu.async_copy(pq_hbm.at[pl.ds(N, N)], qvv, semB)
        b2 = pltpu.async_copy(ei_hbm.at[pl.ds(base, EM)], rv, semB)
        b3 = pltpu.async_copy(ea_hbm.at[pl.ds(base, EM)], eav, semB)

        for k in range(NZ // 16):
            zv[pl.ds(k * 16, 16)] = jnp.zeros((16,), jnp.float32)
        for k in range(8):
            onesv[pl.ds(k * 16, 16)] = jnp.full((16,), 1.0, jnp.float32)
        pltpu.sync_copy(zv, deg_sh.at[pl.ds(s * NZ, NZ)])

        h1.wait()
        plsc.subcore_barrier()

    with jax.named_scope("hist_scatter"):
        def _fire(j, carry):
            pltpu.async_copy(onesv.at[pl.ds(0, 128)],
                             deg_sh.at[cv.at[pl.ds(j * 128, 128)]],
                             add=True, sem=semH)
            return carry

        lax.fori_loop(0, HC, _fire, 0)
        pltpu.async_copy(onesv.at[pl.ds(0, HT)],
                         deg_sh.at[cv.at[pl.ds(HC * 128, HT)]],
                         add=True, sem=semH)

        def _drain(j, carry):
            pltpu.make_async_copy(onesv.at[pl.ds(0, 128)],
                                  deg_sh.at[cv.at[pl.ds(j * 128, 128)]],
                                  semH).wait()
            return carry

        lax.fori_loop(0, HC, _drain, 0)
        pltpu.make_async_copy(onesv.at[pl.ds(0, HT)],
                              deg_sh.at[cv.at[pl.ds(HC * 128, HT)]],
                              semH).wait()
        plsc.subcore_barrier()

    with jax.named_scope("exchange"):
        @pl.when(s == 0)
        def _():
            pltpu.sync_copy(deg_sh, degp_hbm.at[pl.ds(c * NS, NS)])
            pltpu.core_barrier(bsem, core_axis_name="c")

        plsc.subcore_barrier()
        d2 = pltpu.async_copy(degp_hbm.at[pl.ds((1 - c) * NS, NS)], d2v, semA)
        pltpu.sync_copy(deg_sh, degv)
        d2.wait()
        a1.wait()

    with jax.named_scope("table"):
        @plsc.parallel_loop(0, N, step=16, unroll=4)
        def _table(i):
            sl = pl.ds(i, 16)
            d = degv[sl] + d2v[sl]
            half = d * 0.5
            y = lax.bitcast_convert_type(
                jnp.int32(0x5F3759DF)
                - (lax.bitcast_convert_type(d, jnp.int32) >> 1),
                jnp.float32)
            y = y * (1.5 - half * y * y)
            y = y * (1.5 - half * y * y)
            y = y * (1.5 - half * y * y)
            u = jnp.where(d == 0.0, jnp.full((16,), jnp.inf, jnp.float32), y)
            uv[sl] = u
            pcv[sl] = u * pvv[sl]

    with jax.named_scope("wait_edges"):
        b1.wait()
        b2.wait()
        b3.wait()

    with jax.named_scope("edges"):
        @plsc.parallel_loop(0, EM, step=16, unroll=4)
        def _edge(i):
            sl = pl.ds(i, 16)
            ir = rv[sl]
            ic = cv[sl]
            ur = plsc.load_gather(uv, [ir])
            qr = plsc.load_gather(qvv, [ir])
            pcc = plsc.load_gather(pcv, [ic])
            ov[sl] = eav[sl] * (ur * pcc + qr)

    with jax.named_scope("wb"):
        pltpu.sync_copy(ov, out_hbm.at[pl.ds(base, EM)])


_sc_fused = pl.kernel(
    _sc_body,
    out_type=[jax.ShapeDtypeStruct((E,), jnp.float32),
              jax.ShapeDtypeStruct((2 * NS,), jnp.float32)],
    mesh=_MESH,
    compiler_params=_SC_PARAMS,
    scratch_types=[
        pltpu.VMEM((NZ,), jnp.float32),
        pltpu.VMEM((128,), jnp.float32),
        pltpu.VMEM((NS,), jnp.float32),
        pltpu.VMEM((NS,), jnp.float32),
        pltpu.VMEM((N,), jnp.float32),
        pltpu.VMEM((N,), jnp.float32),
        pltpu.VMEM((N,), jnp.float32),
        pltpu.VMEM((N,), jnp.float32),
        pltpu.VMEM((EM,), jnp.int32),
        pltpu.VMEM((EM,), jnp.int32),
        pltpu.VMEM((EM,), jnp.float32),
        pltpu.VMEM((EM,), jnp.float32),
        pltpu.VMEM_SHARED((NS,), jnp.float32),
        pltpu.SemaphoreType.DMA,
        pltpu.SemaphoreType.DMA,
        pltpu.SemaphoreType.DMA,
        pltpu.SemaphoreType.REGULAR,
    ],
)


def _mv_body(x_ref, pw_ref, qw_ref, pb_ref, qb_ref, ei_ref,
             o_ref, oe_ref, oei_ref):
    w = jnp.concatenate([pw_ref[...], qw_ref[...]], axis=0)
    b = jnp.concatenate([pb_ref[...], qb_ref[...]], axis=0)
    res = jnp.maximum(
        lax.dot_general(w, x_ref[...], (((1,), (1,)), ((), ())),
                        preferred_element_type=jnp.float32) + b, 0.0)
    o_ref[pl.ds(0, N)] = res[0, :]
    o_ref[pl.ds(N, N)] = res[1, :]
    oe_ref[pl.ds(0, E)] = ei_ref[0, :]
    oe_ref[pl.ds(E, E)] = ei_ref[1, :]
    oei_ref[...] = ei_ref[...]


def _prep(x, p_w, q_w, p_b, q_b, ei):
    return pl.pallas_call(
        _mv_body,
        out_shape=[jax.ShapeDtypeStruct((2 * N,), jnp.float32),
                   jax.ShapeDtypeStruct((2 * E,), jnp.int32),
                   jax.ShapeDtypeStruct((2, E), jnp.int32)],
    )(x, p_w, q_w, p_b, q_b, ei)


def kernel(x, edge_index, edge_attr, p_w, p_b, q_w, q_b):
    ei = edge_index.astype(jnp.int32)
    pq, eif, ei_out = _prep(x, p_w, q_w, p_b.reshape(1, 1),
                            q_b.reshape(1, 1), ei)
    out, _ = _sc_fused(eif, edge_attr, pq)
    return (ei_out, out)


# speedup vs baseline: 1.8457x; 1.0347x over previous
import jax
import jax.numpy as jnp
from jax import lax
from jax.experimental import pallas as pl
from jax.experimental.pallas import tpu as pltpu
from jax.experimental.pallas import tpu_sc as plsc

N = 10000
E = 320000
D = 128
NS = 10240
EM = E // 32
HC = EM // 128
HT = EM - HC * 128
NZ = NS // 16

_MESH = plsc.VectorSubcoreMesh(core_axis_name="c", subcore_axis_name="s")
_SC_PARAMS = pltpu.CompilerParams(needs_layout_passes=False)


def _sc_body(ei_hbm, ea_hbm, pq_hbm, out_hbm, degp_hbm,
             zv, onesv, degv, d2v, pvv, qvv, uv, pcv, rv, cv, eav, ov,
             deg_sh, semH, semA, semB, bsem):
    c = lax.axis_index("c")
    s = lax.axis_index("s")
    wid = c * 16 + s
    base = wid * EM

    with jax.named_scope("stage"):
        h1 = pltpu.async_copy(ei_hbm.at[pl.ds(E + base, EM)], cv, semH)

        for k in range(NZ // 16):
            zv[pl.ds(k * 16, 16)] = jnp.zeros((16,), jnp.float32)
        for k in range(8):
            onesv[pl.ds(k * 16, 16)] = jnp.full((16,), 1.0, jnp.float32)
        pltpu.sync_copy(zv, deg_sh.at[pl.ds(s * NZ, NZ)])

        h1.wait()
        plsc.subcore_barrier()

    with jax.named_scope("hist_scatter"):
        def _fire(j, carry):
            pltpu.async_copy(onesv.at[pl.ds(0, 128)],
                             deg_sh.at[cv.at[pl.ds(j * 128, 128)]],
                             add=True, sem=semH)
            return carry

        lax.fori_loop(0, HC, _fire, 0)
        pltpu.async_copy(onesv.at[pl.ds(0, HT)],
                         deg_sh.at[cv.at[pl.ds(HC * 128, HT)]],
                         add=True, sem=semH)

        a1 = pltpu.async_copy(pq_hbm.at[pl.ds(0, N)], pvv, semA)
        b1 = pltpu.async_copy(pq_hbm.at[pl.ds(N, N)], qvv, semB)
        b2 = pltpu.async_copy(ei_hbm.at[pl.ds(base, EM)], rv, semB)
        b3 = pltpu.async_copy(ea_hbm.at[pl.ds(base, EM)], eav, semB)

        def _drain(j, carry):
            pltpu.make_async_copy(onesv.at[pl.ds(0, 128)],
                                  deg_sh.at[cv.at[pl.ds(j * 128, 128)]],
                                  semH).wait()
            return carry

        lax.fori_loop(0, HC, _drain, 0)
        pltpu.make_async_copy(onesv.at[pl.ds(0, HT)],
                              deg_sh.at[cv.at[pl.ds(HC * 128, HT)]],
                              semH).wait()
        plsc.subcore_barrier()

    with jax.named_scope("exchange"):
        @pl.when(s == 0)
        def _():
            pltpu.sync_copy(deg_sh, degp_hbm.at[pl.ds(c * NS, NS)])
            pltpu.core_barrier(bsem, core_axis_name="c")

        plsc.subcore_barrier()
        d2 = pltpu.async_copy(degp_hbm.at[pl.ds((1 - c) * NS, NS)], d2v, semA)
        pltpu.sync_copy(deg_sh, degv)
        d2.wait()
        a1.wait()

    with jax.named_scope("table"):
        @plsc.parallel_loop(0, N, step=16, unroll=4)
        def _table(i):
            sl = pl.ds(i, 16)
            d = degv[sl] + d2v[sl]
            half = d * 0.5
            y = lax.bitcast_convert_type(
                jnp.int32(0x5F3759DF)
                - (lax.bitcast_convert_type(d, jnp.int32) >> 1),
                jnp.float32)
            y = y * (1.5 - half * y * y)
            y = y * (1.5 - half * y * y)
            y = y * (1.5 - half * y * y)
            u = jnp.where(d == 0.0, jnp.full((16,), jnp.inf, jnp.float32), y)
            uv[sl] = u
            pcv[sl] = u * pvv[sl]

    with jax.named_scope("wait_edges"):
        b1.wait()
        b2.wait()
        b3.wait()

    with jax.named_scope("edges"):
        @plsc.parallel_loop(0, EM, step=16, unroll=4)
        def _edge(i):
            sl = pl.ds(i, 16)
            ir = rv[sl]
            ic = cv[sl]
            ur = plsc.load_gather(uv, [ir])
            qr = plsc.load_gather(qvv, [ir])
            pcc = plsc.load_gather(pcv, [ic])
            ov[sl] = eav[sl] * (ur * pcc + qr)

    with jax.named_scope("wb"):
        pltpu.sync_copy(ov, out_hbm.at[pl.ds(base, EM)])


_sc_fused = pl.kernel(
    _sc_body,
    out_type=[jax.ShapeDtypeStruct((E,), jnp.float32),
              jax.ShapeDtypeStruct((2 * NS,), jnp.float32)],
    mesh=_MESH,
    compiler_params=_SC_PARAMS,
    scratch_types=[
        pltpu.VMEM((NZ,), jnp.float32),
        pltpu.VMEM((128,), jnp.float32),
        pltpu.VMEM((NS,), jnp.float32),
        pltpu.VMEM((NS,), jnp.float32),
        pltpu.VMEM((N,), jnp.float32),
        pltpu.VMEM((N,), jnp.float32),
        pltpu.VMEM((N,), jnp.float32),
        pltpu.VMEM((N,), jnp.float32),
        pltpu.VMEM((EM,), jnp.int32),
        pltpu.VMEM((EM,), jnp.int32),
        pltpu.VMEM((EM,), jnp.float32),
        pltpu.VMEM((EM,), jnp.float32),
        pltpu.VMEM_SHARED((NS,), jnp.float32),
        pltpu.SemaphoreType.DMA,
        pltpu.SemaphoreType.DMA,
        pltpu.SemaphoreType.DMA,
        pltpu.SemaphoreType.REGULAR,
    ],
)


def _mv_body(x_ref, pw_ref, qw_ref, pb_ref, qb_ref, ei_ref,
             o_ref, oe_ref, oei_ref):
    w = jnp.concatenate([pw_ref[...], qw_ref[...]], axis=0)
    b = jnp.concatenate([pb_ref[...], qb_ref[...]], axis=0)
    res = jnp.maximum(
        lax.dot_general(w, x_ref[...], (((1,), (1,)), ((), ())),
                        preferred_element_type=jnp.float32) + b, 0.0)
    o_ref[pl.ds(0, N)] = res[0, :]
    o_ref[pl.ds(N, N)] = res[1, :]
    oe_ref[pl.ds(0, E)] = ei_ref[0, :]
    oe_ref[pl.ds(E, E)] = ei_ref[1, :]
    oei_ref[...] = ei_ref[...]


def _prep(x, p_w, q_w, p_b, q_b, ei):
    return pl.pallas_call(
        _mv_body,
        out_shape=[jax.ShapeDtypeStruct((2 * N,), jnp.float32),
                   jax.ShapeDtypeStruct((2 * E,), jnp.int32),
                   jax.ShapeDtypeStruct((2, E), jnp.int32)],
    )(x, p_w, q_w, p_b, q_b, ei)


def kernel(x, edge_index, edge_attr, p_w, p_b, q_w, q_b):
    ei = edge_index.astype(jnp.int32)
    pq, eif, ei_out = _prep(x, p_w, q_w, p_b.reshape(1, 1),
                            q_b.reshape(1, 1), ei)
    out, _ = _sc_fused(eif, edge_attr, pq)
    return (ei_out, out)


# speedup vs baseline: 1.8864x; 1.0220x over previous
import jax
import jax.numpy as jnp
from jax import lax
from jax.experimental import pallas as pl
from jax.experimental.pallas import tpu as pltpu
from jax.experimental.pallas import tpu_sc as plsc

N = 10000
E = 320000
D = 128
NS = 10240
NB = E // 128
NBMAX = 79
EW = NBMAX * 128
NZ = NS // 16

_MESH = plsc.VectorSubcoreMesh(core_axis_name="c", subcore_axis_name="s")
_SC_PARAMS = pltpu.CompilerParams(needs_layout_passes=False)


def _sc_body(ei_hbm, ea_hbm, pq_hbm, out_hbm, degp_hbm,
             zv, onesv, degv, d2v, pvv, qvv, uv, pcv, rcv, eav, ov,
             deg_sh, semH, semA, semB, bsem):
    c = lax.axis_index("c")
    s = lax.axis_index("s")
    wid = c * 16 + s
    bb = wid * 78 + jnp.maximum(wid - 28, 0)
    nblk = jnp.where(wid >= 28, NBMAX, 78)
    base = bb * 128

    with jax.named_scope("stage"):
        h1 = pltpu.async_copy(ei_hbm.at[:, pl.ds(base, EW)], rcv, semH)

        for k in range(NZ // 16):
            zv[pl.ds(k * 16, 16)] = jnp.zeros((16,), jnp.float32)
        for k in range(8):
            onesv[pl.ds(k * 16, 16)] = jnp.full((16,), 1.0, jnp.float32)
        pltpu.sync_copy(zv, deg_sh.at[pl.ds(s * NZ, NZ)])

        h1.wait()
        plsc.subcore_barrier()

    with jax.named_scope("hist_scatter"):
        def _fire(j, carry):
            pltpu.async_copy(onesv.at[pl.ds(0, 128)],
                             deg_sh.at[rcv.at[1, pl.ds(j * 128, 128)]],
                             add=True, sem=semH)
            return carry

        lax.fori_loop(0, nblk, _fire, 0)

        a1 = pltpu.async_copy(pq_hbm.at[pl.ds(0, N)], pvv, semA)
        b1 = pltpu.async_copy(pq_hbm.at[pl.ds(N, N)], qvv, semB)
        b2 = pltpu.async_copy(ea_hbm.at[pl.ds(base, EW)], eav, semB)

        def _drain(j, carry):
            pltpu.make_async_copy(onesv.at[pl.ds(0, 128)],
                                  deg_sh.at[rcv.at[1, pl.ds(j * 128, 128)]],
                                  semH).wait()
            return carry

        lax.fori_loop(0, nblk, _drain, 0)
        plsc.subcore_barrier()

    with jax.named_scope("exchange"):
        @pl.when(s == 0)
        def _():
            pltpu.sync_copy(deg_sh, degp_hbm.at[pl.ds(c * NS, NS)])
            pltpu.core_barrier(bsem, core_axis_name="c")

        plsc.subcore_barrier()
        d2 = pltpu.async_copy(degp_hbm.at[pl.ds((1 - c) * NS, NS)], d2v, semA)
        pltpu.sync_copy(deg_sh, degv)
        d2.wait()
        a1.wait()

    with jax.named_scope("table"):
        @plsc.parallel_loop(0, N, step=16, unroll=4)
        def _table(i):
            sl = pl.ds(i, 16)
            d = degv[sl] + d2v[sl]
            half = d * 0.5
            y = lax.bitcast_convert_type(
                jnp.int32(0x5F3759DF)
                - (lax.bitcast_convert_type(d, jnp.int32) >> 1),
                jnp.float32)
            y = y * (1.5 - half * y * y)
            y = y * (1.5 - half * y * y)
            y = y * (1.5 - half * y * y)
            u = jnp.where(d == 0.0, jnp.full((16,), jnp.inf, jnp.float32), y)
            uv[sl] = u
            pcv[sl] = u * pvv[sl]

    with jax.named_scope("wait_edges"):
        b1.wait()
        b2.wait()

    with jax.named_scope("edges"):
        @plsc.parallel_loop(0, EW, step=16, unroll=4)
        def _edge(i):
            sl = pl.ds(i, 16)
            ir = rcv[0, sl]
            ic = rcv[1, sl]
            ur = plsc.load_gather(uv, [ir])
            qr = plsc.load_gather(qvv, [ir])
            pcc = plsc.load_gather(pcv, [ic])
            ov[sl] = eav[sl] * (ur * pcc + qr)

    with jax.named_scope("wb"):
        pltpu.sync_copy(ov.at[pl.ds(0, 78 * 128)],
                        out_hbm.at[pl.ds(base, 78 * 128)])

        @pl.when(wid >= 28)
        def _():
            pltpu.sync_copy(ov.at[pl.ds(78 * 128, 128)],
                            out_hbm.at[pl.ds(base + 78 * 128, 128)])


_sc_fused = pl.kernel(
    _sc_body,
    out_type=[jax.ShapeDtypeStruct((E,), jnp.float32),
              jax.ShapeDtypeStruct((2 * NS,), jnp.float32)],
    mesh=_MESH,
    compiler_params=_SC_PARAMS,
    scratch_types=[
        pltpu.VMEM((NZ,), jnp.float32),
        pltpu.VMEM((128,), jnp.float32),
        pltpu.VMEM((NS,), jnp.float32),
        pltpu.VMEM((NS,), jnp.float32),
        pltpu.VMEM((N,), jnp.float32),
        pltpu.VMEM((N,), jnp.float32),
        pltpu.VMEM((N,), jnp.float32),
        pltpu.VMEM((N,), jnp.float32),
        pltpu.VMEM((2, EW), jnp.int32),
        pltpu.VMEM((EW,), jnp.float32),
        pltpu.VMEM((EW,), jnp.float32),
        pltpu.VMEM_SHARED((NS,), jnp.float32),
        pltpu.SemaphoreType.DMA,
        pltpu.SemaphoreType.DMA,
        pltpu.SemaphoreType.DMA,
        pltpu.SemaphoreType.REGULAR,
    ],
)


def _mv_body(x_ref, pw_ref, qw_ref, pb_ref, qb_ref, o_ref):
    w = jnp.concatenate([pw_ref[...], qw_ref[...]], axis=0)
    b = jnp.concatenate([pb_ref[...], qb_ref[...]], axis=0)
    res = jnp.maximum(
        lax.dot_general(w, x_ref[...], (((1,), (1,)), ((), ())),
                        preferred_element_type=jnp.float32) + b, 0.0)
    o_ref[pl.ds(0, N)] = res[0, :]
    o_ref[pl.ds(N, N)] = res[1, :]


def _matvec(x, p_w, q_w, p_b, q_b):
    return pl.pallas_call(
        _mv_body,
        out_shape=jax.ShapeDtypeStruct((2 * N,), jnp.float32),
    )(x, p_w, q_w, p_b, q_b)


def kernel(x, edge_index, edge_attr, p_w, p_b, q_w, q_b):
    ei = edge_index.astype(jnp.int32)
    pq = _matvec(x, p_w, q_w, p_b.reshape(1, 1), q_b.reshape(1, 1))
    out, _ = _sc_fused(ei, edge_attr, pq)
    return (edge_index, out)
